# baseline (speedup 1.0000x reference)
import functools

import jax
import jax.numpy as jnp
from jax import lax
from jax.experimental import pallas as pl
from jax.experimental.pallas import tpu as pltpu
from jax.experimental.pallas import tpu_sc as plsc

_MIN_SIZE = 0.01
_MAX_SIZE = 0.999
_NW = 32
_CH = 8000
_FULL = 2_000_000
_NCHUNK = _FULL // _CH
_BASE_CHUNKS = _NCHUNK // _NW
_EXTRA = _NCHUNK % _NW


def _normalize_chunk(buf, sv):
    lanes = lax.iota(jnp.int32, 16)
    even0 = lanes * 2

    def blk(i, carry):
        ie = even0 + i * 32
        io = ie + 1
        a = plsc.load_gather(buf, [ie])
        b = plsc.load_gather(buf, [io])
        t = a * a + b * b
        bits = plsc.bitcast(t, jnp.int32)
        bits = 0x5F3759DF - lax.shift_right_logical(bits, 1)
        y = plsc.bitcast(bits, jnp.float32)
        y = y * (1.5 - 0.5 * t * y * y)
        y = y * (1.5 - 0.5 * t * y * y)
        norm = t * y
        factor = jnp.where(norm >= 1e-12, sv * y, sv * 1e12)
        plsc.store_scatter(buf, [ie], a * factor)
        plsc.store_scatter(buf, [io], b * factor)
        return carry

    lax.fori_loop(0, _CH // 32, blk, 0)


def _make_sc_call():
    mesh = plsc.VectorSubcoreMesh(core_axis_name="c", subcore_axis_name="s")

    @functools.partial(
        pl.kernel,
        out_type=jax.ShapeDtypeStruct((_FULL,), jnp.float32),
        mesh=mesh,
        scratch_types=[
            pltpu.VMEM((_CH,), jnp.float32),
            pltpu.VMEM((16,), jnp.float32),
        ],
        compiler_params=pltpu.CompilerParams(needs_layout_passes=False),
    )
    def run(w_hbm, s_hbm, out_hbm, buf, sbuf):
        wid = lax.axis_index("s") * 2 + lax.axis_index("c")
        pltpu.sync_copy(s_hbm, sbuf)
        sv = jnp.clip(sbuf[...], _MIN_SIZE, _MAX_SIZE)
        nchunks = jnp.where(wid < _EXTRA, _BASE_CHUNKS + 1, _BASE_CHUNKS)

        def chunk_body(j, carry):
            cid = j * _NW + wid
            off = pl.multiple_of(cid * _CH, _CH)
            pltpu.sync_copy(w_hbm.at[pl.ds(off, _CH)], buf)
            _normalize_chunk(buf, sv)
            pltpu.sync_copy(buf, out_hbm.at[pl.ds(off, _CH)])
            return carry

        lax.fori_loop(0, nchunks, chunk_body, 0)

    return run


_sc_call = _make_sc_call()


def kernel(weight, scale):
    flat = weight.reshape(_FULL)
    s16 = jnp.broadcast_to(scale, (16,))
    out = _sc_call(flat, s16)
    return out.reshape(weight.shape)
